# Initial kernel scaffold; baseline (speedup 1.0000x reference)
import jax
import jax.numpy as jnp
from jax.experimental import pallas as pl


def kernel(batch_feature_src, batch_feature_tgt, global_avg_weights, global_max_weights, ns_src, ns_tgt, adjacency_matrixs, W_cross, W_gat, a_gat):
    raise NotImplementedError("write your pallas kernel here")


# speedup vs baseline: 1.9736x; 1.9736x over previous
import functools
import math

import jax
import jax.numpy as jnp
from jax.experimental import pallas as pl
from jax.experimental.pallas import tpu as pltpu


def _cross_body(fs_ref, ft_ref, wc_ref, wcat_ref, a12_ref,
                whs_ref, wht_ref, es_ref, et_ref, mean_ref, max_ref, *, scale):
    fs = fs_ref[0]
    ft = ft_ref[0]
    wc = wc_ref[...]
    hs = jnp.dot(fs, wc, preferred_element_type=jnp.float32)
    ht = jnp.dot(ft, wc, preferred_element_type=jnp.float32)
    sim = jax.lax.dot_general(hs, ht, (((1,), (1,)), ((), ())),
                              preferred_element_type=jnp.float32) * scale
    m1 = jnp.max(sim, axis=1, keepdims=True)
    p1 = jnp.exp(sim - m1)
    a_st = p1 / jnp.sum(p1, axis=1, keepdims=True)
    att_src = jnp.dot(a_st, ft, preferred_element_type=jnp.float32)
    m0 = jnp.max(sim, axis=0, keepdims=True)
    p0 = jnp.exp(sim - m0)
    a_ts = p0 / jnp.sum(p0, axis=0, keepdims=True)
    att_tgt = jax.lax.dot_general(a_ts, fs, (((0,), (0,)), ((), ())),
                                  preferred_element_type=jnp.float32)
    emb_s = jnp.concatenate([fs - att_src, fs], axis=1)
    emb_t = jnp.concatenate([ft - att_tgt, ft], axis=1)
    wcat = wcat_ref[...]
    wh_s = jnp.dot(emb_s, wcat, preferred_element_type=jnp.float32)
    wh_t = jnp.dot(emb_t, wcat, preferred_element_type=jnp.float32)
    whs_ref[0] = wh_s
    wht_ref[0] = wh_t
    a12 = a12_ref[...]
    es_ref[0] = jnp.dot(wh_s, a12, preferred_element_type=jnp.float32)
    et_ref[0] = jnp.dot(wh_t, a12, preferred_element_type=jnp.float32)
    mean_ref[...] = jnp.full((1, 8, 128), jnp.mean(sim), dtype=jnp.float32)
    max_ref[...] = jnp.full((1, 8, 128), jnp.max(sim), dtype=jnp.float32)


def _gat_body(adj_ref, es_ref, edt_ref, wh_ref, orig_ref, out_ref,
              *, nheads, nhid):
    adj = adj_ref[...]
    wh = wh_ref[...]
    es = es_ref[...]
    edt = edt_ref[...]
    for h in range(nheads):
        e = es[:, h][:, None] + edt[h + nheads, :][None, :]
        e = jnp.where(e >= 0, e, 0.2 * e)
        e = jnp.where(adj > 0, e, jnp.float32(-9e15))
        m = jnp.max(e, axis=1, keepdims=True)
        p = jnp.exp(e - m)
        s = jnp.sum(p, axis=1, keepdims=True)
        num = jnp.dot(p, wh[:, h * nhid:(h + 1) * nhid],
                      preferred_element_type=jnp.float32)
        upd = num / s
        upd = jnp.where(upd > 0, upd, jnp.exp(jnp.minimum(upd, 0.0)) - 1.0)
        out_ref[:, h * nhid:(h + 1) * nhid] = (
            orig_ref[:, h * nhid:(h + 1) * nhid] - upd)


def kernel(batch_feature_src, batch_feature_tgt, global_avg_weights,
           global_max_weights, ns_src, ns_tgt, adjacency_matrixs,
           W_cross, W_gat, a_gat):
    B, N, D = batch_feature_src.shape
    NHEADS, twoD, NHID = W_gat.shape
    HD = NHEADS * NHID
    TOTAL = 2 * B * N
    scale = 1.0 / math.sqrt(D)

    W_cat = jnp.transpose(W_gat, (1, 0, 2)).reshape(twoD, HD)
    eye = jnp.eye(NHEADS, dtype=jnp.float32)
    A1 = (eye[:, None, :] * a_gat[:, :NHID, None]).reshape(HD, NHEADS)
    A2 = (eye[:, None, :] * a_gat[:, NHID:, None]).reshape(HD, NHEADS)
    A12 = jnp.concatenate([A1, A2], axis=1)

    wh_s, wh_t, e_s, e_t, meanb, maxb = pl.pallas_call(
        functools.partial(_cross_body, scale=scale),
        grid=(B,),
        in_specs=[
            pl.BlockSpec((1, N, D), lambda i: (i, 0, 0)),
            pl.BlockSpec((1, N, D), lambda i: (i, 0, 0)),
            pl.BlockSpec((D, D), lambda i: (0, 0)),
            pl.BlockSpec((twoD, HD), lambda i: (0, 0)),
            pl.BlockSpec((HD, 2 * NHEADS), lambda i: (0, 0)),
        ],
        out_specs=[
            pl.BlockSpec((1, N, HD), lambda i: (i, 0, 0)),
            pl.BlockSpec((1, N, HD), lambda i: (i, 0, 0)),
            pl.BlockSpec((1, N, 2 * NHEADS), lambda i: (i, 0, 0)),
            pl.BlockSpec((1, N, 2 * NHEADS), lambda i: (i, 0, 0)),
            pl.BlockSpec((1, 8, 128), lambda i: (i, 0, 0)),
            pl.BlockSpec((1, 8, 128), lambda i: (i, 0, 0)),
        ],
        out_shape=[
            jax.ShapeDtypeStruct((B, N, HD), jnp.float32),
            jax.ShapeDtypeStruct((B, N, HD), jnp.float32),
            jax.ShapeDtypeStruct((B, N, 2 * NHEADS), jnp.float32),
            jax.ShapeDtypeStruct((B, N, 2 * NHEADS), jnp.float32),
            jax.ShapeDtypeStruct((B, 8, 128), jnp.float32),
            jax.ShapeDtypeStruct((B, 8, 128), jnp.float32),
        ],
    )(batch_feature_src, batch_feature_tgt, W_cross, W_cat, A12)

    cross_attention = (global_avg_weights * meanb[:, 0, 0]
                       + global_max_weights * maxb[:, 0, 0])

    Wh = jnp.stack([wh_s, wh_t], axis=1).reshape(TOTAL, HD)
    E = jnp.stack([e_s, e_t], axis=1).reshape(TOTAL, 2 * NHEADS)
    EDT = E.T
    orig = jnp.stack([batch_feature_src, batch_feature_tgt],
                     axis=1).reshape(TOTAL, D)

    BLK = 512
    out_node = pl.pallas_call(
        functools.partial(_gat_body, nheads=NHEADS, nhid=NHID),
        grid=(TOTAL // BLK,),
        in_specs=[
            pl.BlockSpec((BLK, TOTAL), lambda i: (i, 0)),
            pl.BlockSpec((BLK, 2 * NHEADS), lambda i: (i, 0)),
            pl.BlockSpec((2 * NHEADS, TOTAL), lambda i: (0, 0)),
            pl.BlockSpec((TOTAL, HD), lambda i: (0, 0)),
            pl.BlockSpec((BLK, D), lambda i: (i, 0)),
        ],
        out_specs=pl.BlockSpec((BLK, D), lambda i: (i, 0)),
        out_shape=jax.ShapeDtypeStruct((TOTAL, D), jnp.float32),
    )(adjacency_matrixs, E, EDT, Wh, orig)

    ns = jnp.stack([ns_src, ns_tgt], axis=1).reshape(-1).astype(jnp.int32)
    return cross_attention, out_node, ns


# speedup vs baseline: 1.9888x; 1.0077x over previous
import functools
import math

import jax
import jax.numpy as jnp
from jax.experimental import pallas as pl
from jax.experimental.pallas import tpu as pltpu


def _cross_body(fs_ref, ft_ref, wc_ref, wcat_ref, a12_ref,
                whs_ref, wht_ref, es_ref, et_ref, mean_ref, max_ref, *, scale):
    fs = fs_ref[0]
    ft = ft_ref[0]
    wc = wc_ref[...]
    hs = jnp.dot(fs, wc, preferred_element_type=jnp.float32)
    ht = jnp.dot(ft, wc, preferred_element_type=jnp.float32)
    sim = jax.lax.dot_general(hs, ht, (((1,), (1,)), ((), ())),
                              preferred_element_type=jnp.float32) * scale
    m1 = jnp.max(sim, axis=1, keepdims=True)
    p1 = jnp.exp(sim - m1)
    a_st = p1 / jnp.sum(p1, axis=1, keepdims=True)
    att_src = jnp.dot(a_st, ft, preferred_element_type=jnp.float32)
    m0 = jnp.max(sim, axis=0, keepdims=True)
    p0 = jnp.exp(sim - m0)
    a_ts = p0 / jnp.sum(p0, axis=0, keepdims=True)
    att_tgt = jax.lax.dot_general(a_ts, fs, (((0,), (0,)), ((), ())),
                                  preferred_element_type=jnp.float32)
    emb_s = jnp.concatenate([fs - att_src, fs], axis=1)
    emb_t = jnp.concatenate([ft - att_tgt, ft], axis=1)
    wcat = wcat_ref[...]
    wh_s = jnp.dot(emb_s, wcat, preferred_element_type=jnp.float32)
    wh_t = jnp.dot(emb_t, wcat, preferred_element_type=jnp.float32)
    whs_ref[0] = wh_s
    wht_ref[0] = wh_t
    a12 = a12_ref[...]
    es_ref[0] = jnp.dot(wh_s, a12, preferred_element_type=jnp.float32)
    et_ref[0] = jnp.dot(wh_t, a12, preferred_element_type=jnp.float32)
    mean_ref[...] = jnp.full((1, 8, 128), jnp.mean(sim), dtype=jnp.float32)
    max_ref[...] = jnp.full((1, 8, 128), jnp.max(sim), dtype=jnp.float32)


def _gat_body(adj_ref, es_ref, edt_ref, wh_ref, orig_ref, out_ref,
              *, nheads, nhid):
    badj = adj_ref[...] > 0
    wh = wh_ref[...]
    es = es_ref[...]
    edt = edt_ref[...]
    for h in range(nheads):
        t = es[:, h][:, None] + edt[h + nheads, :][None, :]
        e = jnp.maximum(t, 0.2 * t)
        e = jnp.where(badj, e, jnp.float32(-9e15))
        m = jnp.max(e, axis=1, keepdims=True)
        p = jnp.exp(e - m)
        s = jnp.sum(p, axis=1, keepdims=True)
        num = jnp.dot(p, wh[:, h * nhid:(h + 1) * nhid],
                      preferred_element_type=jnp.float32)
        upd = num / s
        upd = jnp.where(upd > 0, upd, jnp.exp(jnp.minimum(upd, 0.0)) - 1.0)
        out_ref[:, h * nhid:(h + 1) * nhid] = (
            orig_ref[:, h * nhid:(h + 1) * nhid] - upd)


def kernel(batch_feature_src, batch_feature_tgt, global_avg_weights,
           global_max_weights, ns_src, ns_tgt, adjacency_matrixs,
           W_cross, W_gat, a_gat):
    B, N, D = batch_feature_src.shape
    NHEADS, twoD, NHID = W_gat.shape
    HD = NHEADS * NHID
    TOTAL = 2 * B * N
    scale = 1.0 / math.sqrt(D)

    W_cat = jnp.transpose(W_gat, (1, 0, 2)).reshape(twoD, HD)
    eye = jnp.eye(NHEADS, dtype=jnp.float32)
    A1 = (eye[:, None, :] * a_gat[:, :NHID, None]).reshape(HD, NHEADS)
    A2 = (eye[:, None, :] * a_gat[:, NHID:, None]).reshape(HD, NHEADS)
    A12 = jnp.concatenate([A1, A2], axis=1)

    wh_s, wh_t, e_s, e_t, meanb, maxb = pl.pallas_call(
        functools.partial(_cross_body, scale=scale),
        grid=(B,),
        in_specs=[
            pl.BlockSpec((1, N, D), lambda i: (i, 0, 0)),
            pl.BlockSpec((1, N, D), lambda i: (i, 0, 0)),
            pl.BlockSpec((D, D), lambda i: (0, 0)),
            pl.BlockSpec((twoD, HD), lambda i: (0, 0)),
            pl.BlockSpec((HD, 2 * NHEADS), lambda i: (0, 0)),
        ],
        out_specs=[
            pl.BlockSpec((1, N, HD), lambda i: (i, 0, 0)),
            pl.BlockSpec((1, N, HD), lambda i: (i, 0, 0)),
            pl.BlockSpec((1, N, 2 * NHEADS), lambda i: (i, 0, 0)),
            pl.BlockSpec((1, N, 2 * NHEADS), lambda i: (i, 0, 0)),
            pl.BlockSpec((1, 8, 128), lambda i: (i, 0, 0)),
            pl.BlockSpec((1, 8, 128), lambda i: (i, 0, 0)),
        ],
        out_shape=[
            jax.ShapeDtypeStruct((B, N, HD), jnp.float32),
            jax.ShapeDtypeStruct((B, N, HD), jnp.float32),
            jax.ShapeDtypeStruct((B, N, 2 * NHEADS), jnp.float32),
            jax.ShapeDtypeStruct((B, N, 2 * NHEADS), jnp.float32),
            jax.ShapeDtypeStruct((B, 8, 128), jnp.float32),
            jax.ShapeDtypeStruct((B, 8, 128), jnp.float32),
        ],
    )(batch_feature_src, batch_feature_tgt, W_cross, W_cat, A12)

    cross_attention = (global_avg_weights * meanb[:, 0, 0]
                       + global_max_weights * maxb[:, 0, 0])

    Wh = jnp.stack([wh_s, wh_t], axis=1).reshape(TOTAL, HD)
    E = jnp.stack([e_s, e_t], axis=1).reshape(TOTAL, 2 * NHEADS)
    EDT = E.T
    orig = jnp.stack([batch_feature_src, batch_feature_tgt],
                     axis=1).reshape(TOTAL, D)

    BLK = 512
    out_node = pl.pallas_call(
        functools.partial(_gat_body, nheads=NHEADS, nhid=NHID),
        grid=(TOTAL // BLK,),
        in_specs=[
            pl.BlockSpec((BLK, TOTAL), lambda i: (i, 0)),
            pl.BlockSpec((BLK, 2 * NHEADS), lambda i: (i, 0)),
            pl.BlockSpec((2 * NHEADS, TOTAL), lambda i: (0, 0)),
            pl.BlockSpec((TOTAL, HD), lambda i: (0, 0)),
            pl.BlockSpec((BLK, D), lambda i: (i, 0)),
        ],
        out_specs=pl.BlockSpec((BLK, D), lambda i: (i, 0)),
        out_shape=jax.ShapeDtypeStruct((TOTAL, D), jnp.float32),
    )(adjacency_matrixs, E, EDT, Wh, orig)

    ns = jnp.stack([ns_src, ns_tgt], axis=1).reshape(-1).astype(jnp.int32)
    return cross_attention, out_node, ns


# speedup vs baseline: 2.1856x; 1.0990x over previous
import functools
import math

import jax
import jax.numpy as jnp
from jax.experimental import pallas as pl
from jax.experimental.pallas import tpu as pltpu


def _cross_body(fs_ref, ft_ref, wc_ref, wcat_ref, a12_ref,
                whs_ref, wht_ref, es_ref, et_ref, mean_ref, max_ref, *, scale):
    fs = fs_ref[0]
    ft = ft_ref[0]
    wc = wc_ref[...]
    hs = jnp.dot(fs, wc, preferred_element_type=jnp.float32)
    ht = jnp.dot(ft, wc, preferred_element_type=jnp.float32)
    sim = jax.lax.dot_general(hs, ht, (((1,), (1,)), ((), ())),
                              preferred_element_type=jnp.float32) * scale
    m1 = jnp.max(sim, axis=1, keepdims=True)
    p1 = jnp.exp(sim - m1)
    a_st = p1 / jnp.sum(p1, axis=1, keepdims=True)
    att_src = jnp.dot(a_st, ft, preferred_element_type=jnp.float32)
    m0 = jnp.max(sim, axis=0, keepdims=True)
    p0 = jnp.exp(sim - m0)
    a_ts = p0 / jnp.sum(p0, axis=0, keepdims=True)
    att_tgt = jax.lax.dot_general(a_ts, fs, (((0,), (0,)), ((), ())),
                                  preferred_element_type=jnp.float32)
    emb_s = jnp.concatenate([fs - att_src, fs], axis=1)
    emb_t = jnp.concatenate([ft - att_tgt, ft], axis=1)
    wcat = wcat_ref[...]
    wh_s = jnp.dot(emb_s, wcat, preferred_element_type=jnp.float32)
    wh_t = jnp.dot(emb_t, wcat, preferred_element_type=jnp.float32)
    whs_ref[0] = wh_s
    wht_ref[0] = wh_t
    a12 = a12_ref[...]
    es_ref[0] = jnp.dot(wh_s, a12, preferred_element_type=jnp.float32)
    et_ref[0] = jnp.dot(wh_t, a12, preferred_element_type=jnp.float32)
    mean_ref[...] = jnp.full((1, 8, 128), jnp.mean(sim), dtype=jnp.float32)
    max_ref[...] = jnp.full((1, 8, 128), jnp.max(sim), dtype=jnp.float32)


def _gat_body(adj_ref, es_ref, edt_ref, wh_ref, orig_ref, out_ref,
              *, nheads, nhid):
    adj = adj_ref[...]
    wh = wh_ref[...]
    es = es_ref[...]
    edt = edt_ref[...]
    total = adj.shape[1]
    cm = jnp.sum(wh, axis=0, keepdims=True) * (1.0 / total)
    for h in range(nheads):
        t = es[:, h][:, None] + edt[h + nheads, :][None, :]
        l = jnp.maximum(t, 0.2 * t)
        m = jnp.max(l, axis=1, keepdims=True)
        p = jnp.exp(l - m) * adj
        s = jnp.sum(p, axis=1, keepdims=True)
        num = jnp.dot(p, wh[:, h * nhid:(h + 1) * nhid],
                      preferred_element_type=jnp.float32)
        pos = s > 0
        upd = jnp.where(pos, num / jnp.where(pos, s, 1.0),
                        cm[:, h * nhid:(h + 1) * nhid])
        upd = jnp.where(upd > 0, upd, jnp.exp(jnp.minimum(upd, 0.0)) - 1.0)
        out_ref[:, h * nhid:(h + 1) * nhid] = (
            orig_ref[:, h * nhid:(h + 1) * nhid] - upd)


def kernel(batch_feature_src, batch_feature_tgt, global_avg_weights,
           global_max_weights, ns_src, ns_tgt, adjacency_matrixs,
           W_cross, W_gat, a_gat):
    B, N, D = batch_feature_src.shape
    NHEADS, twoD, NHID = W_gat.shape
    HD = NHEADS * NHID
    TOTAL = 2 * B * N
    scale = 1.0 / math.sqrt(D)

    W_cat = jnp.transpose(W_gat, (1, 0, 2)).reshape(twoD, HD)
    eye = jnp.eye(NHEADS, dtype=jnp.float32)
    A1 = (eye[:, None, :] * a_gat[:, :NHID, None]).reshape(HD, NHEADS)
    A2 = (eye[:, None, :] * a_gat[:, NHID:, None]).reshape(HD, NHEADS)
    A12 = jnp.concatenate([A1, A2], axis=1)

    wh_s, wh_t, e_s, e_t, meanb, maxb = pl.pallas_call(
        functools.partial(_cross_body, scale=scale),
        grid=(B,),
        in_specs=[
            pl.BlockSpec((1, N, D), lambda i: (i, 0, 0)),
            pl.BlockSpec((1, N, D), lambda i: (i, 0, 0)),
            pl.BlockSpec((D, D), lambda i: (0, 0)),
            pl.BlockSpec((twoD, HD), lambda i: (0, 0)),
            pl.BlockSpec((HD, 2 * NHEADS), lambda i: (0, 0)),
        ],
        out_specs=[
            pl.BlockSpec((1, N, HD), lambda i: (i, 0, 0)),
            pl.BlockSpec((1, N, HD), lambda i: (i, 0, 0)),
            pl.BlockSpec((1, N, 2 * NHEADS), lambda i: (i, 0, 0)),
            pl.BlockSpec((1, N, 2 * NHEADS), lambda i: (i, 0, 0)),
            pl.BlockSpec((1, 8, 128), lambda i: (i, 0, 0)),
            pl.BlockSpec((1, 8, 128), lambda i: (i, 0, 0)),
        ],
        out_shape=[
            jax.ShapeDtypeStruct((B, N, HD), jnp.float32),
            jax.ShapeDtypeStruct((B, N, HD), jnp.float32),
            jax.ShapeDtypeStruct((B, N, 2 * NHEADS), jnp.float32),
            jax.ShapeDtypeStruct((B, N, 2 * NHEADS), jnp.float32),
            jax.ShapeDtypeStruct((B, 8, 128), jnp.float32),
            jax.ShapeDtypeStruct((B, 8, 128), jnp.float32),
        ],
    )(batch_feature_src, batch_feature_tgt, W_cross, W_cat, A12)

    cross_attention = (global_avg_weights * meanb[:, 0, 0]
                       + global_max_weights * maxb[:, 0, 0])

    Wh = jnp.stack([wh_s, wh_t], axis=1).reshape(TOTAL, HD)
    E = jnp.stack([e_s, e_t], axis=1).reshape(TOTAL, 2 * NHEADS)
    EDT = E.T
    orig = jnp.stack([batch_feature_src, batch_feature_tgt],
                     axis=1).reshape(TOTAL, D)

    BLK = 512
    out_node = pl.pallas_call(
        functools.partial(_gat_body, nheads=NHEADS, nhid=NHID),
        grid=(TOTAL // BLK,),
        in_specs=[
            pl.BlockSpec((BLK, TOTAL), lambda i: (i, 0)),
            pl.BlockSpec((BLK, 2 * NHEADS), lambda i: (i, 0)),
            pl.BlockSpec((2 * NHEADS, TOTAL), lambda i: (0, 0)),
            pl.BlockSpec((TOTAL, HD), lambda i: (0, 0)),
            pl.BlockSpec((BLK, D), lambda i: (i, 0)),
        ],
        out_specs=pl.BlockSpec((BLK, D), lambda i: (i, 0)),
        out_shape=jax.ShapeDtypeStruct((TOTAL, D), jnp.float32),
    )(adjacency_matrixs, E, EDT, Wh, orig)

    ns = jnp.stack([ns_src, ns_tgt], axis=1).reshape(-1).astype(jnp.int32)
    return cross_attention, out_node, ns
